# baseline (speedup 1.0000x reference)
import jax
import jax.numpy as jnp
from jax import lax
from jax.experimental import pallas as pl
from jax.experimental.pallas import tpu as pltpu
from jax.experimental.pallas import tpu_sc as plsc

_N = 320000
_T = 10000
_D = 128
_W = 144
_BLK = 128
_NB = _N // _BLK
_TP = 10112
_RPT = _TP // 16
_TRASH = 10100

_MAGIC = 0x5F3759DF


def _rsqrt16(x):
    i = plsc.bitcast(x, jnp.int32)
    i = _MAGIC - lax.shift_right_logical(i, 1)
    y = plsc.bitcast(i, jnp.float32)
    for _ in range(3):
        y = y * (1.5 - 0.5 * x * y * y)
    return y


def _splat_sum16(x, lane):
    for b in (1, 2, 4, 8):
        x = x + x[lane ^ b]
    return x


def _sc_body(raw_hbm, idx_hbm, out_hbm,
             idx_v, buf_a, buf_b, cstage, cidx, table,
             sem_ra, sem_rb, sem_ji, sem_f):
    c = lax.axis_index("c")
    s = lax.axis_index("s")
    wid = c * 16 + s
    lane = lax.iota(jnp.int32, 16)
    zero16 = jnp.zeros((16,), jnp.float32)
    flag16 = jnp.where(lane == 0, 1.0, 0.0).astype(jnp.float32)
    shp = jnp.minimum(lane + 1, 15)

    def zrow(r, _):
        for k in range(9):
            cstage[r, pl.ds(k * 16, 16)] = zero16
        return 0

    lax.fori_loop(0, 32, zrow, 0)
    base_t = s * _RPT
    for t in range(19):
        pltpu.sync_copy(cstage.at[pl.ds(0, 32)],
                        table.at[pl.ds(base_t + t * 32, 32)])
    pltpu.sync_copy(cstage.at[pl.ds(0, 24)],
                    table.at[pl.ds(base_t + 608, 24)])
    plsc.subcore_barrier()

    start_blk = 78 * wid + jnp.minimum(wid, 4)
    pltpu.sync_copy(idx_hbm.at[pl.ds(start_blk, 4)], idx_v)

    def process_block(buf, irow, next0, carry):
        def group16(g, carry):
            a_regs, cc, kp, nrun = carry[:9], carry[9], carry[10], carry[11]
            base = g * 16
            idv = idx_v[irow, pl.ds(base, 16)]
            idn = idx_v[irow, pl.ds(jnp.minimum(base + 16, 112), 16)]
            nf = jnp.where(g == 7, next0, idn[0])
            sh = idv[shp]
            sh = jnp.where(lane == 15, jnp.full((16,), nf, jnp.int32), sh)
            ends = (idv != sh).astype(jnp.int32)
            for r in range(16):
                row = base + r
                vs = [buf[row, pl.ds(k * 16, 16)] for k in range(8)]
                acc = vs[0] * vs[0]
                for k in range(1, 8):
                    acc = acc + vs[k] * vs[k]
                x = _splat_sum16(acc, lane)
                y = _rsqrt16(x)
                new_a = [a_regs[k] * kp + vs[k] * y for k in range(8)]
                new_a.append(a_regs[8] * kp + flag16)
                a_regs = new_a
                e = ends[r]
                ee = e != 0
                sl = nrun & 31
                cc = jnp.where(lane == (nrun & 15),
                               jnp.full((16,), idv[r], jnp.int32), cc)
                nr2 = nrun + e

                @pl.when(ee)
                def _store():
                    for k in range(9):
                        cstage[sl, pl.ds(k * 16, 16)] = a_regs[k]

                @pl.when(ee & ((nr2 & 15) == 0))
                def _flush():
                    crow = 1 - ((nr2 >> 4) & 1)

                    @pl.when(nr2 >= 32)
                    def _wait_prev():
                        pltpu.make_async_copy(
                            cstage.at[pl.ds((1 - crow) * 16, 16)],
                            table.at[cidx.at[1 - crow]], sem_f).wait()

                    cidx[crow, pl.ds(0, 16)] = cc
                    pltpu.async_copy(cstage.at[pl.ds(crow * 16, 16)],
                                     table.at[cidx.at[crow]], sem_f, add=True)

                kf = jnp.where(ee, 0.0, 1.0)
                kp = jnp.full((16,), kf, jnp.float32)
                nrun = nr2
            return a_regs + [cc, kp, nrun]

        return lax.fori_loop(0, 8, group16, carry)

    carry = ([zero16] * 9
             + [jnp.full((16,), _TRASH, jnp.int32),
                jnp.full((16,), 1.0, jnp.float32),
                jnp.int32(0)])

    def pair(p, carry):
        ia = 2 * (p & 1)
        ib = ia + 1
        inx = 2 - ia
        b0 = start_blk + 2 * p
        pfetch = pltpu.async_copy(
            idx_hbm.at[pl.ds(jnp.minimum(b0 + 2, _NB - 2), 2)],
            idx_v.at[pl.ds(inx, 2)], sem_ji)
        in_a = pltpu.async_copy(raw_hbm.at[pl.ds(b0 * _BLK, _BLK)],
                                buf_a, sem_ra)
        in_b = pltpu.async_copy(raw_hbm.at[pl.ds((b0 + 1) * _BLK, _BLK)],
                                buf_b, sem_rb)
        in_a.wait()
        next0_a = idx_v[ib, pl.ds(0, 16)][0]
        carry = process_block(buf_a, ia, next0_a, carry)
        pfetch.wait()
        in_b.wait()
        is_last = jnp.logical_and(p == 38, wid >= 4)
        next0_b = jnp.where(is_last, -1, idx_v[inx, pl.ds(0, 16)][0])
        carry = process_block(buf_b, ib, next0_b, carry)
        return carry

    carry = lax.fori_loop(0, 39, pair, carry)

    def final_flush(wn, cc):
        @pl.when((wn & 15) != 0)
        def _resid():
            crow = (wn >> 4) & 1
            ccp = jnp.where(lane < (wn & 15), cc,
                            jnp.full((16,), _TRASH, jnp.int32))

            @pl.when(wn >= 16)
            def _wait_prev():
                pltpu.make_async_copy(
                    cstage.at[pl.ds((1 - crow) * 16, 16)],
                    table.at[cidx.at[1 - crow]], sem_f).wait()

            cidx[crow, pl.ds(0, 16)] = ccp
            pltpu.sync_copy(cstage.at[pl.ds(crow * 16, 16)],
                            table.at[cidx.at[crow]], add=True)

        @pl.when(((wn & 15) == 0) & (wn >= 16))
        def _drain():
            crowp = 1 - ((wn >> 4) & 1)
            pltpu.make_async_copy(
                cstage.at[pl.ds(crowp * 16, 16)],
                table.at[cidx.at[crowp]], sem_f).wait()

    @pl.when(wid < 4)
    def _tail():
        b = start_blk + 78
        pltpu.sync_copy(raw_hbm.at[pl.ds(b * _BLK, _BLK)], buf_a)
        carry2 = process_block(buf_a, 2, jnp.int32(-1), carry)
        final_flush(carry2[11], carry2[9])

    @pl.when(wid >= 4)
    def _main_flush():
        final_flush(carry[11], carry[9])

    plsc.subcore_barrier()
    pltpu.sync_copy(table.at[pl.ds(base_t, _RPT)],
                    out_hbm.at[c, pl.ds(base_t, _RPT)])


_sc_scatter = pl.kernel(
    _sc_body,
    out_type=jax.ShapeDtypeStruct((2, _TP, _W), jnp.float32),
    mesh=plsc.VectorSubcoreMesh(core_axis_name="c", subcore_axis_name="s"),
    compiler_params=pltpu.CompilerParams(use_tc_tiling_on_sc=False,
                                         needs_layout_passes=False),
    scratch_types=[
        pltpu.VMEM((4, _BLK), jnp.int32),
        pltpu.VMEM((_BLK, _D), jnp.float32),
        pltpu.VMEM((_BLK, _D), jnp.float32),
        pltpu.VMEM((32, _W), jnp.float32),
        pltpu.VMEM((2, 16), jnp.int32),
        pltpu.VMEM_SHARED((_TP, _W), jnp.float32),
        pltpu.SemaphoreType.DMA,
        pltpu.SemaphoreType.DMA,
        pltpu.SemaphoreType.DMA,
        pltpu.SemaphoreType.DMA,
    ],
)


def _epilogue_body(sp_ref, t_ref, o_ref):
    sp = sp_ref[...]
    t = t_ref[0] + t_ref[1]
    seg_sum = t[:_T, :_D]
    counts = t[:_T, _D:_D + 1]

    ss = jnp.sum(sp * sp, axis=1, keepdims=True)
    spn = sp / jnp.maximum(jnp.sqrt(ss), 1e-12)

    cc = jnp.maximum(counts, 1.0)
    mean = seg_sum / cc

    dot = jnp.sum(spn * mean, axis=1)
    na = jnp.maximum(jnp.sqrt(jnp.sum(spn * spn, axis=1)), 1e-8)
    nb = jnp.maximum(jnp.sqrt(jnp.sum(mean * mean, axis=1)), 1e-8)
    cos = dot / (na * nb)
    weights = counts[:, 0] / float(_N)
    o_ref[...] = jnp.sum((1.0 - cos) * weights).reshape(1, 1)


def _epilogue_tc(sp, tables):
    return pl.pallas_call(
        _epilogue_body,
        out_shape=jax.ShapeDtypeStruct((1, 1), jnp.float32),
    )(sp, tables)


def kernel(superPoint_feat, rawPoint_feat, point_assignment):
    idx = point_assignment.reshape(_NB, _BLK)
    tables = _sc_scatter(rawPoint_feat, idx)
    loss = _epilogue_tc(superPoint_feat, tables)
    return loss[0, 0]
